# baseline (speedup 1.0000x reference)
import functools

import jax
import jax.numpy as jnp
from jax.experimental import pallas as pl
from jax.experimental.pallas import tpu as pltpu


def _silu(y):
    return y * (1.0 / (1.0 + jnp.exp(-y)))


def _downc_kernel(h, w, x_ref, w1_ref, b1_ref, w2_ref, b2_ref, w3_ref, b3_ref,
                  o_ref, xt_ref, y1_ref):
    c1 = x_ref.shape[2]
    c_ = w1_ref.shape[1]
    c2h = w2_ref.shape[1]
    ho, wo = h // 2, w // 2
    p = ho * wo
    nch = c1 // 128

    xb = x_ref[0]
    for c in range(nch):
        xt_ref[c] = xb[:, c * 128:(c + 1) * 128].astype(jnp.float32) \
                      .reshape(h, w, 128)

    y1 = jnp.dot(xb, w1_ref[...], preferred_element_type=jnp.float32)
    y1 = _silu(y1 + b1_ref[...])

    y1_ref[7:8, :, :] = jnp.zeros((1, w + 8, c_), jnp.float32)
    y1_ref[:, 0:8, :] = jnp.zeros((h + 8, 8, c_), jnp.float32)
    y1_ref[8:h + 8, 8:w + 8, :] = y1.reshape(h, w, c_)

    taps = []
    for kh in range(3):
        for kw in range(3):
            t = y1_ref[pl.ds(kh + 7, ho, 2), pl.ds(kw + 7, wo, 2), :]
            taps.append(t.reshape(p, c_).astype(jnp.bfloat16))
    patches = jnp.concatenate(taps, axis=1)
    y2 = jax.lax.dot_general(w2_ref[...], patches, (((0,), (1,)), ((), ())),
                             preferred_element_type=jnp.float32)
    y2 = _silu(y2 + b2_ref[...])
    o_ref[0, 0:c2h, :, :] = y2.astype(o_ref.dtype).reshape(c2h, ho, wo)

    y3 = None
    for c in range(nch):
        p00 = xt_ref[c, pl.ds(0, ho, 2), pl.ds(0, wo, 2), :]
        p01 = xt_ref[c, pl.ds(0, ho, 2), pl.ds(1, wo, 2), :]
        p10 = xt_ref[c, pl.ds(1, ho, 2), pl.ds(0, wo, 2), :]
        p11 = xt_ref[c, pl.ds(1, ho, 2), pl.ds(1, wo, 2), :]
        xm = jnp.maximum(jnp.maximum(p00, p01), jnp.maximum(p10, p11))
        xm = xm.astype(jnp.bfloat16).reshape(p, 128)
        part = jax.lax.dot_general(w3_ref[c * 128:(c + 1) * 128, :], xm,
                                   (((0,), (1,)), ((), ())),
                                   preferred_element_type=jnp.float32)
        y3 = part if y3 is None else y3 + part
    y3 = _silu(y3 + b3_ref[...])
    o_ref[0, c2h:2 * c2h, :, :] = y3.astype(o_ref.dtype).reshape(c2h, ho, wo)


def kernel(x, w1, s1, b1, w2, s2, b2, w3, s3, b3):
    n, c1, h, w = x.shape
    c_ = w1.shape[0]
    c2h = w2.shape[0]
    ho, wo = h // 2, w // 2
    p = ho * wo

    xt3 = jnp.transpose(x.reshape(n, c1, h * w), (0, 2, 1))
    xt3 = xt3.astype(jnp.bfloat16)

    w1s = (w1.reshape(c_, c1) * s1[:, None]).T.astype(jnp.bfloat16)
    b1r = b1.reshape(1, c_).astype(jnp.float32)
    w2s = (jnp.transpose(w2, (2, 3, 1, 0)) * s2).reshape(9 * c_, c2h)
    w2s = w2s.astype(jnp.bfloat16)
    b2c = b2.reshape(c2h, 1).astype(jnp.float32)
    w3s = (w3.reshape(c2h, c1) * s3[:, None]).T.astype(jnp.bfloat16)
    b3c = b3.reshape(c2h, 1).astype(jnp.float32)

    body = functools.partial(_downc_kernel, h, w)

    out = pl.pallas_call(
        body,
        out_shape=jax.ShapeDtypeStruct((n, 2 * c2h, ho, wo), x.dtype),
        grid=(n,),
        in_specs=[
            pl.BlockSpec((1, h * w, c1), lambda i: (i, 0, 0)),
            pl.BlockSpec((c1, c_), lambda i: (0, 0)),
            pl.BlockSpec((1, c_), lambda i: (0, 0)),
            pl.BlockSpec((9 * c_, c2h), lambda i: (0, 0)),
            pl.BlockSpec((c2h, 1), lambda i: (0, 0)),
            pl.BlockSpec((c1, c2h), lambda i: (0, 0)),
            pl.BlockSpec((c2h, 1), lambda i: (0, 0)),
        ],
        out_specs=pl.BlockSpec((1, 2 * c2h, ho, wo), lambda i: (i, 0, 0, 0)),
        scratch_shapes=[
            pltpu.VMEM((c1 // 128, h, w, 128), jnp.float32),
            pltpu.VMEM((h + 8, w + 8, c_), jnp.float32),
        ],
        compiler_params=pltpu.CompilerParams(
            dimension_semantics=("parallel",)),
    )(xt3, w1s, b1r, w2s, b2c, w3s, b3c)

    return out


# speedup vs baseline: 1.3236x; 1.3236x over previous
import functools

import jax
import jax.numpy as jnp
from jax.experimental import pallas as pl
from jax.experimental.pallas import tpu as pltpu


def _silu(y):
    return y * (1.0 / (1.0 + jnp.exp(-y)))


def _downc_kernel(h, w, x_ref, w1_ref, b1_ref, w2_ref, b2_ref, w3_ref, b3_ref,
                  o_ref, xt_ref, y1_ref):
    c1 = x_ref.shape[2]
    c_ = w1_ref.shape[1]
    c2h = w2_ref.shape[1]
    ho, wo = h // 2, w // 2
    p = ho * wo
    nch = c1 // 128

    xb = x_ref[0]

    @pl.when(pl.program_id(1) == 0)
    def _cv1_cv2():
        y1 = jnp.dot(xb, w1_ref[...], preferred_element_type=jnp.float32)
        y1 = _silu(y1 + b1_ref[...])

        y1_ref[7:8, :, :] = jnp.zeros((1, w + 8, c_), jnp.float32)
        y1_ref[:, 0:8, :] = jnp.zeros((h + 8, 8, c_), jnp.float32)
        y1_ref[8:h + 8, 8:w + 8, :] = y1.reshape(h, w, c_)

        taps = []
        for kh in range(3):
            for kw in range(3):
                t = y1_ref[pl.ds(kh + 7, ho, 2), pl.ds(kw + 7, wo, 2), :]
                taps.append(t.reshape(p, c_).astype(jnp.bfloat16))
        patches = jnp.concatenate(taps, axis=1)
        y2 = jax.lax.dot_general(w2_ref[...], patches,
                                 (((0,), (1,)), ((), ())),
                                 preferred_element_type=jnp.float32)
        y2 = _silu(y2 + b2_ref[...])
        o_ref[0, :, :] = y2.astype(o_ref.dtype)

    @pl.when(pl.program_id(1) == 1)
    def _cv3():
        for c in range(nch):
            xt_ref[c] = xb[:, c * 128:(c + 1) * 128].astype(jnp.float32) \
                          .reshape(h, w, 128)
        y3 = None
        for c in range(nch):
            p00 = xt_ref[c, pl.ds(0, ho, 2), pl.ds(0, wo, 2), :]
            p01 = xt_ref[c, pl.ds(0, ho, 2), pl.ds(1, wo, 2), :]
            p10 = xt_ref[c, pl.ds(1, ho, 2), pl.ds(0, wo, 2), :]
            p11 = xt_ref[c, pl.ds(1, ho, 2), pl.ds(1, wo, 2), :]
            xm = jnp.maximum(jnp.maximum(p00, p01), jnp.maximum(p10, p11))
            xm = xm.astype(jnp.bfloat16).reshape(p, 128)
            part = jax.lax.dot_general(w3_ref[c * 128:(c + 1) * 128, :], xm,
                                       (((0,), (1,)), ((), ())),
                                       preferred_element_type=jnp.float32)
            y3 = part if y3 is None else y3 + part
        y3 = _silu(y3 + b3_ref[...])
        o_ref[0, :, :] = y3.astype(o_ref.dtype)


def kernel(x, w1, s1, b1, w2, s2, b2, w3, s3, b3):
    n, c1, h, w = x.shape
    c_ = w1.shape[0]
    c2h = w2.shape[0]
    ho, wo = h // 2, w // 2
    p = ho * wo

    xt3 = jnp.transpose(x.reshape(n, c1, h * w), (0, 2, 1))
    xt3 = xt3.astype(jnp.bfloat16)

    w1s = (w1.reshape(c_, c1) * s1[:, None]).T.astype(jnp.bfloat16)
    b1r = b1.reshape(1, c_).astype(jnp.float32)
    w2s = (jnp.transpose(w2, (2, 3, 1, 0)) * s2).reshape(9 * c_, c2h)
    w2s = w2s.astype(jnp.bfloat16)
    b2c = b2.reshape(c2h, 1).astype(jnp.float32)
    w3s = (w3.reshape(c2h, c1) * s3[:, None]).T.astype(jnp.bfloat16)
    b3c = b3.reshape(c2h, 1).astype(jnp.float32)

    body = functools.partial(_downc_kernel, h, w)

    out = pl.pallas_call(
        body,
        out_shape=jax.ShapeDtypeStruct((n, 2 * c2h, p), x.dtype),
        grid=(n, 2),
        in_specs=[
            pl.BlockSpec((1, h * w, c1), lambda i, j: (i, 0, 0)),
            pl.BlockSpec((c1, c_), lambda i, j: (0, 0)),
            pl.BlockSpec((1, c_), lambda i, j: (0, 0)),
            pl.BlockSpec((9 * c_, c2h), lambda i, j: (0, 0)),
            pl.BlockSpec((c2h, 1), lambda i, j: (0, 0)),
            pl.BlockSpec((c1, c2h), lambda i, j: (0, 0)),
            pl.BlockSpec((c2h, 1), lambda i, j: (0, 0)),
        ],
        out_specs=pl.BlockSpec((1, c2h, p), lambda i, j: (i, j, 0)),
        scratch_shapes=[
            pltpu.VMEM((c1 // 128, h, w, 128), jnp.float32),
            pltpu.VMEM((h + 8, w + 8, c_), jnp.float32),
        ],
        compiler_params=pltpu.CompilerParams(
            dimension_semantics=("parallel", "arbitrary")),
    )(xt3, w1s, b1r, w2s, b2c, w3s, b3c)

    return out.reshape(n, 2 * c2h, ho, wo)


# speedup vs baseline: 1.6648x; 1.2579x over previous
import functools

import jax
import jax.numpy as jnp
from jax.experimental import pallas as pl
from jax.experimental.pallas import tpu as pltpu


def _silu(y):
    return y * (1.0 / (1.0 + jnp.exp(-y)))


def _downc_kernel(h, w, x_ref, w1_ref, b1_ref, w2_ref, b2_ref, w3_ref, b3_ref,
                  o_ref, xt_ref, y1_ref):
    c1 = x_ref.shape[2]
    c_ = w1_ref.shape[1]
    c2h = w2_ref.shape[1]
    ho, wo = h // 2, w // 2
    p = ho * wo
    nch = c1 // 128

    for img in range(x_ref.shape[0]):
        xb = x_ref[img]

        y1 = jnp.dot(xb, w1_ref[...], preferred_element_type=jnp.float32)
        y1 = _silu(y1 + b1_ref[...])

        y1_ref[7:8, :, :] = jnp.zeros((1, w + 8, c_), jnp.float32)
        y1_ref[:, 0:8, :] = jnp.zeros((h + 8, 8, c_), jnp.float32)
        y1_ref[8:h + 8, 8:w + 8, :] = y1.reshape(h, w, c_)

        taps = []
        for kh in range(3):
            for kw in range(3):
                t = y1_ref[pl.ds(kh + 7, ho, 2), pl.ds(kw + 7, wo, 2), :]
                taps.append(t.reshape(p, c_).astype(jnp.bfloat16))
        patches = jnp.concatenate(taps, axis=1)
        y2 = jax.lax.dot_general(w2_ref[...], patches,
                                 (((0,), (1,)), ((), ())),
                                 preferred_element_type=jnp.float32)
        y2 = _silu(y2 + b2_ref[...])
        o_ref[img, 0:c2h, :] = y2.astype(o_ref.dtype)

        for c in range(nch):
            xt_ref[c] = xb[:, c * 128:(c + 1) * 128].astype(jnp.float32) \
                          .reshape(h, w, 128)
        y3 = None
        for c in range(nch):
            p00 = xt_ref[c, pl.ds(0, ho, 2), pl.ds(0, wo, 2), :]
            p01 = xt_ref[c, pl.ds(0, ho, 2), pl.ds(1, wo, 2), :]
            p10 = xt_ref[c, pl.ds(1, ho, 2), pl.ds(0, wo, 2), :]
            p11 = xt_ref[c, pl.ds(1, ho, 2), pl.ds(1, wo, 2), :]
            xm = jnp.maximum(jnp.maximum(p00, p01), jnp.maximum(p10, p11))
            xm = xm.astype(jnp.bfloat16).reshape(p, 128)
            part = jax.lax.dot_general(w3_ref[c * 128:(c + 1) * 128, :], xm,
                                       (((0,), (1,)), ((), ())),
                                       preferred_element_type=jnp.float32)
            y3 = part if y3 is None else y3 + part
        y3 = _silu(y3 + b3_ref[...])
        o_ref[img, c2h:2 * c2h, :] = y3.astype(o_ref.dtype)


def kernel(x, w1, s1, b1, w2, s2, b2, w3, s3, b3):
    n, c1, h, w = x.shape
    c_ = w1.shape[0]
    c2h = w2.shape[0]
    ho, wo = h // 2, w // 2
    p = ho * wo

    xt3 = jnp.transpose(x.reshape(n, c1, h * w), (0, 2, 1))
    xt3 = xt3.astype(jnp.bfloat16)

    w1s = (w1.reshape(c_, c1) * s1[:, None]).T.astype(jnp.bfloat16)
    b1r = b1.reshape(1, c_).astype(jnp.float32)
    w2s = (jnp.transpose(w2, (2, 3, 1, 0)) * s2).reshape(9 * c_, c2h)
    w2s = w2s.astype(jnp.bfloat16)
    b2c = b2.reshape(c2h, 1).astype(jnp.float32)
    w3s = (w3.reshape(c2h, c1) * s3[:, None]).T.astype(jnp.bfloat16)
    b3c = b3.reshape(c2h, 1).astype(jnp.float32)

    body = functools.partial(_downc_kernel, h, w)

    out = pl.pallas_call(
        body,
        out_shape=jax.ShapeDtypeStruct((n, 2 * c2h, p), x.dtype),
        grid=(n // 2,),
        in_specs=[
            pl.BlockSpec((2, h * w, c1), lambda i: (i, 0, 0)),
            pl.BlockSpec((c1, c_), lambda i: (0, 0)),
            pl.BlockSpec((1, c_), lambda i: (0, 0)),
            pl.BlockSpec((9 * c_, c2h), lambda i: (0, 0)),
            pl.BlockSpec((c2h, 1), lambda i: (0, 0)),
            pl.BlockSpec((c1, c2h), lambda i: (0, 0)),
            pl.BlockSpec((c2h, 1), lambda i: (0, 0)),
        ],
        out_specs=pl.BlockSpec((2, 2 * c2h, p), lambda i: (i, 0, 0)),
        scratch_shapes=[
            pltpu.VMEM((c1 // 128, h, w, 128), jnp.float32),
            pltpu.VMEM((h + 8, w + 8, c_), jnp.float32),
        ],
        compiler_params=pltpu.CompilerParams(
            dimension_semantics=("parallel",)),
    )(xt3, w1s, b1r, w2s, b2c, w3s, b3c)

    return out.reshape(n, 2 * c2h, ho, wo)


# speedup vs baseline: 1.8067x; 1.0852x over previous
import functools

import jax
import jax.numpy as jnp
from jax.experimental import pallas as pl
from jax.experimental.pallas import tpu as pltpu


def _silu(y):
    return y * (1.0 / (1.0 + jnp.exp(-y)))


def _downc_kernel(h, w, x_ref, w1_ref, b1_ref, w2_ref, b2_ref, w3_ref, b3_ref,
                  o_ref, xt_ref, y1_ref):
    c1 = x_ref.shape[2]
    c_ = w1_ref.shape[1]
    c2h = w2_ref.shape[1]
    ho, wo = h // 2, w // 2
    p = ho * wo
    nch = c1 // 128

    for img in range(1):
        xb = x_ref[img]

        y1 = jnp.dot(xb, w1_ref[...], preferred_element_type=jnp.float32)
        y1 = _silu(y1 + b1_ref[...])

        y1_ref[7:8, :, :] = jnp.zeros((1, w + 8, c_), jnp.float32)
        y1_ref[:, 0:8, :] = jnp.zeros((h + 8, 8, c_), jnp.float32)
        y1_ref[8:h + 8, 8:w + 8, :] = y1.reshape(h, w, c_)

        taps = []
        for kh in range(3):
            for kw in range(3):
                t = y1_ref[pl.ds(kh + 7, ho, 2), pl.ds(kw + 7, wo, 2), :]
                taps.append(t.reshape(p, c_).astype(jnp.bfloat16))
        patches = jnp.concatenate(taps, axis=1)
        y2 = jax.lax.dot_general(w2_ref[...], patches,
                                 (((0,), (1,)), ((), ())),
                                 preferred_element_type=jnp.float32)
        y2 = _silu(y2 + b2_ref[...])
        o_ref[img, 0:c2h, :] = y2.astype(o_ref.dtype)

        for c in range(nch):
            xt_ref[c] = xb[:, c * 128:(c + 1) * 128].astype(jnp.float32) \
                          .reshape(h, w, 128)
        y3 = None
        for c in range(nch):
            p00 = xt_ref[c, pl.ds(0, ho, 2), pl.ds(0, wo, 2), :]
            p01 = xt_ref[c, pl.ds(0, ho, 2), pl.ds(1, wo, 2), :]
            p10 = xt_ref[c, pl.ds(1, ho, 2), pl.ds(0, wo, 2), :]
            p11 = xt_ref[c, pl.ds(1, ho, 2), pl.ds(1, wo, 2), :]
            xm = jnp.maximum(jnp.maximum(p00, p01), jnp.maximum(p10, p11))
            xm = xm.astype(jnp.bfloat16).reshape(p, 128)
            part = jax.lax.dot_general(w3_ref[c * 128:(c + 1) * 128, :], xm,
                                       (((0,), (1,)), ((), ())),
                                       preferred_element_type=jnp.float32)
            y3 = part if y3 is None else y3 + part
        y3 = _silu(y3 + b3_ref[...])
        o_ref[img, c2h:2 * c2h, :] = y3.astype(o_ref.dtype)


def kernel(x, w1, s1, b1, w2, s2, b2, w3, s3, b3):
    n, c1, h, w = x.shape
    c_ = w1.shape[0]
    c2h = w2.shape[0]
    ho, wo = h // 2, w // 2
    p = ho * wo

    xt3 = jnp.transpose(x.reshape(n, c1, h * w), (0, 2, 1))
    xt3 = xt3.astype(jnp.bfloat16)

    w1s = (w1.reshape(c_, c1) * s1[:, None]).T.astype(jnp.bfloat16)
    b1r = b1.reshape(1, c_).astype(jnp.float32)
    w2s = (jnp.transpose(w2, (2, 3, 1, 0)) * s2).reshape(9 * c_, c2h)
    w2s = w2s.astype(jnp.bfloat16)
    b2c = b2.reshape(c2h, 1).astype(jnp.float32)
    w3s = (w3.reshape(c2h, c1) * s3[:, None]).T.astype(jnp.bfloat16)
    b3c = b3.reshape(c2h, 1).astype(jnp.float32)

    body = functools.partial(_downc_kernel, h, w)

    out = pl.pallas_call(
        body,
        out_shape=jax.ShapeDtypeStruct((n, 2 * c2h, p), jnp.bfloat16),
        grid=(n,),
        in_specs=[
            pl.BlockSpec((1, h * w, c1), lambda i: (i, 0, 0)),
            pl.BlockSpec((c1, c_), lambda i: (0, 0)),
            pl.BlockSpec((1, c_), lambda i: (0, 0)),
            pl.BlockSpec((9 * c_, c2h), lambda i: (0, 0)),
            pl.BlockSpec((c2h, 1), lambda i: (0, 0)),
            pl.BlockSpec((c1, c2h), lambda i: (0, 0)),
            pl.BlockSpec((c2h, 1), lambda i: (0, 0)),
        ],
        out_specs=pl.BlockSpec((1, 2 * c2h, p), lambda i: (i, 0, 0)),
        scratch_shapes=[
            pltpu.VMEM((c1 // 128, h, w, 128), jnp.float32),
            pltpu.VMEM((h + 8, w + 8, c_), jnp.float32),
        ],
        compiler_params=pltpu.CompilerParams(
            dimension_semantics=("parallel",)),
    )(xt3, w1s, b1r, w2s, b2c, w3s, b3c)

    return out.reshape(n, 2 * c2h, ho, wo).astype(x.dtype)


# speedup vs baseline: 1.8724x; 1.0364x over previous
import functools

import jax
import jax.numpy as jnp
from jax.experimental import pallas as pl
from jax.experimental.pallas import tpu as pltpu


def _silu(y):
    return y * (1.0 / (1.0 + jnp.exp(-y)))


def _downc_kernel(h, w, x_ref, wp_ref, b1_ref, bp_ref, o_ref, xt_ref, y1_ref):
    c1 = x_ref.shape[2]
    c_ = b1_ref.shape[1]
    c2h = wp_ref.shape[1]
    ho, wo = h // 2, w // 2
    p = ho * wo
    nch = c1 // 128
    w1_ref = wp_ref.at[0:c1, 0:c_]
    w2_ref = wp_ref.at[c1:c1 + 9 * c_, :]
    w3_ref = wp_ref.at[c1 + 9 * c_:2 * c1 + 9 * c_, :]
    b2_ref = bp_ref.at[:, 0:1]
    b3_ref = bp_ref.at[:, 1:2]

    for img in range(1):
        xb = x_ref[img]

        y1 = jnp.dot(xb, w1_ref[...], preferred_element_type=jnp.float32)
        y1 = _silu(y1 + b1_ref[...])

        y1_ref[7:8, :, :] = jnp.zeros((1, w + 8, c_), jnp.float32)
        y1_ref[:, 0:8, :] = jnp.zeros((h + 8, 8, c_), jnp.float32)
        y1_ref[8:h + 8, 8:w + 8, :] = y1.reshape(h, w, c_)

        taps = []
        for kh in range(3):
            for kw in range(3):
                t = y1_ref[pl.ds(kh + 7, ho, 2), pl.ds(kw + 7, wo, 2), :]
                taps.append(t.reshape(p, c_).astype(jnp.bfloat16))
        patches = jnp.concatenate(taps, axis=1)
        y2 = jax.lax.dot_general(w2_ref[...], patches,
                                 (((0,), (1,)), ((), ())),
                                 preferred_element_type=jnp.float32)
        y2 = _silu(y2 + b2_ref[...])
        o_ref[img, 0:c2h, :] = y2.astype(o_ref.dtype)

        for c in range(nch):
            xt_ref[c] = xb[:, c * 128:(c + 1) * 128].astype(jnp.float32) \
                          .reshape(h, w, 128)
        y3 = None
        for c in range(nch):
            p00 = xt_ref[c, pl.ds(0, ho, 2), pl.ds(0, wo, 2), :]
            p01 = xt_ref[c, pl.ds(0, ho, 2), pl.ds(1, wo, 2), :]
            p10 = xt_ref[c, pl.ds(1, ho, 2), pl.ds(0, wo, 2), :]
            p11 = xt_ref[c, pl.ds(1, ho, 2), pl.ds(1, wo, 2), :]
            xm = jnp.maximum(jnp.maximum(p00, p01), jnp.maximum(p10, p11))
            xm = xm.astype(jnp.bfloat16).reshape(p, 128)
            part = jax.lax.dot_general(w3_ref[c * 128:(c + 1) * 128, :], xm,
                                       (((0,), (1,)), ((), ())),
                                       preferred_element_type=jnp.float32)
            y3 = part if y3 is None else y3 + part
        y3 = _silu(y3 + b3_ref[...])
        o_ref[img, c2h:2 * c2h, :] = y3.astype(o_ref.dtype)


def kernel(x, w1, s1, b1, w2, s2, b2, w3, s3, b3):
    n, c1, h, w = x.shape
    c_ = w1.shape[0]
    c2h = w2.shape[0]
    ho, wo = h // 2, w // 2
    p = ho * wo

    xt3 = jnp.transpose(x.reshape(n, c1, h * w), (0, 2, 1))
    xt3 = xt3.astype(jnp.bfloat16)

    w1s = (w1.reshape(c_, c1) * s1[:, None]).T.astype(jnp.bfloat16)
    w1p = jnp.pad(w1s, ((0, 0), (0, c2h - c_)))
    b1r = b1.reshape(1, c_).astype(jnp.float32)
    w2s = (jnp.transpose(w2, (2, 3, 1, 0)) * s2).reshape(9 * c_, c2h)
    w2s = w2s.astype(jnp.bfloat16)
    w3s = (w3.reshape(c2h, c1) * s3[:, None]).T.astype(jnp.bfloat16)
    wpack = jnp.concatenate([w1p, w2s, w3s], axis=0)
    bpack = jnp.stack([b2, b3], axis=1).astype(jnp.float32)

    body = functools.partial(_downc_kernel, h, w)

    out = pl.pallas_call(
        body,
        out_shape=jax.ShapeDtypeStruct((n, 2 * c2h, p), jnp.bfloat16),
        grid=(n,),
        in_specs=[
            pl.BlockSpec((1, h * w, c1), lambda i: (i, 0, 0)),
            pl.BlockSpec((2 * c1 + 9 * c_, c2h), lambda i: (0, 0)),
            pl.BlockSpec((1, c_), lambda i: (0, 0)),
            pl.BlockSpec((c2h, 2), lambda i: (0, 0)),
        ],
        out_specs=pl.BlockSpec((1, 2 * c2h, p), lambda i: (i, 0, 0)),
        scratch_shapes=[
            pltpu.VMEM((c1 // 128, h, w, 128), jnp.float32),
            pltpu.VMEM((h + 8, w + 8, c_), jnp.float32),
        ],
        compiler_params=pltpu.CompilerParams(
            dimension_semantics=("parallel",)),
    )(xt3, wpack, b1r, bpack)

    return out.reshape(n, 2 * c2h, ho, wo).astype(x.dtype)
